# Initial kernel scaffold; baseline (speedup 1.0000x reference)
import jax
import jax.numpy as jnp
from jax.experimental import pallas as pl


def kernel(x):
    raise NotImplementedError("write your pallas kernel here")


# speedup vs baseline: 6.3442x; 6.3442x over previous
import functools

import jax
import jax.numpy as jnp
from jax import lax
from jax.experimental import pallas as pl
from jax.experimental.pallas import tpu as pltpu
from jax.experimental.pallas import tpu_sc as plsc

NC = 2
NS = 16
NW = NC * NS
L = 16
B = 1024


def _sparsemax_rows(x):
    R, N = x.shape
    nvec = N // L
    rows_per_w = R // NW
    nchunk = B // L

    mesh = plsc.VectorSubcoreMesh(core_axis_name="c", subcore_axis_name="s")

    @functools.partial(
        pl.kernel,
        out_type=jax.ShapeDtypeStruct((R, N), jnp.float32),
        mesh=mesh,
        scratch_types=[
            pltpu.VMEM((N,), jnp.float32),
            pltpu.VMEM((B,), jnp.float32),
            pltpu.VMEM((B,), jnp.float32),
        ],
        compiler_params=pltpu.CompilerParams(needs_layout_passes=False),
    )
    def sparsemax_kernel(x_hbm, out_hbm, row_v, cnt_v, sum_v):
        wid = lax.axis_index("s") * NC + lax.axis_index("c")
        zvec = jnp.zeros((L,), jnp.float32)
        ones = jnp.ones((L,), jnp.float32)
        iota_f = lax.iota(jnp.int32, L).astype(jnp.float32)
        rev_iota_f = jnp.float32(L - 1) - iota_f

        def zero_hist(j, carry):
            cnt_v[pl.ds(j * L, L)] = zvec
            sum_v[pl.ds(j * L, L)] = zvec
            return carry

        def scan_round(K0, S0, lo, w):
            def body(t, carry):
                carryC, carryS, predsum, kaddv, saddv = carry
                cb = (nchunk - 1) - t
                c = cnt_v[pl.ds(cb * L, L)]
                s = sum_v[pl.ds(cb * L, L)]
                rc = lax.rev(c, (0,))
                rs = lax.rev(s, (0,))
                csum = plsc.cumsum(rc) + carryC
                ssum = plsc.cumsum(rs) + carryS
                base = (cb * L).astype(jnp.float32)
                edges = lo + w * (base + rev_iota_f)
                f = (S0 + ssum) - (K0 + csum) * edges
                pred = jnp.where(f >= 1.0, 1.0, 0.0)
                npred = 1.0 - pred
                return (carryC + jnp.sum(rc), carryS + jnp.sum(rs),
                        predsum + pred, kaddv + rc * npred,
                        saddv + rs * npred)

            init = (jnp.float32(0.0), jnp.float32(0.0), zvec, zvec, zvec)
            _, _, predsum, kaddv, saddv = lax.fori_loop(
                0, nchunk, body, init)
            jstar = jnp.maximum(jnp.sum(predsum) - 1.0, 0.0)
            return jstar, K0 + jnp.sum(kaddv), S0 + jnp.sum(saddv)

        def process_row(r):
            pltpu.sync_copy(x_hbm.at[r], row_v)

            def max_body(j, acc):
                return jnp.maximum(acc, row_v[pl.ds(j * L, L)])
            m = jnp.max(lax.fori_loop(
                0, nvec, max_body, jnp.full((L,), -jnp.inf, jnp.float32)))

            lo1 = m - 1.0
            span1 = 1.0 + 1.0 / B
            inv_w1 = B / span1
            w1 = span1 / B
            lax.fori_loop(0, nchunk, zero_hist, 0)

            def h1_body(j, carry):
                v = row_v[pl.ds(j * L, L)]
                ji = jnp.clip(((v - lo1) * inv_w1).astype(jnp.int32),
                              0, B - 1)
                msk = v >= lo1
                plsc.addupdate_scatter(cnt_v, [ji], ones, mask=msk)
                plsc.addupdate_scatter(sum_v, [ji], v, mask=msk)
                return carry
            lax.fori_loop(0, nvec, h1_body, 0)

            jstar1, K1, S1 = scan_round(
                jnp.float32(0.0), jnp.float32(0.0), lo1, w1)
            jstar1_i = jstar1.astype(jnp.int32)

            lo2 = lo1 + jstar1 * w1
            inv_w2 = jnp.float32(B) / w1
            w2 = w1 / B
            lax.fori_loop(0, nchunk, zero_hist, 0)

            def h2_body(j, carry):
                v = row_v[pl.ds(j * L, L)]
                j1 = jnp.clip(((v - lo1) * inv_w1).astype(jnp.int32),
                              0, B - 1)
                msk = (v >= lo1) & (j1 == jstar1_i)
                j2 = jnp.clip(((v - lo2) * inv_w2).astype(jnp.int32),
                              0, B - 1)
                plsc.addupdate_scatter(cnt_v, [j2], ones, mask=msk)
                plsc.addupdate_scatter(sum_v, [j2], v, mask=msk)
                return carry
            lax.fori_loop(0, nvec, h2_body, 0)

            jstar2, K2, S2 = scan_round(K1, S1, lo2, w2)

            lo_f = lo2 + jstar2 * w2
            tau = jnp.clip((S2 + zvec - 1.0) / jnp.maximum(K2 + zvec, 1.0),
                           lo_f, lo_f + w2)
            tau = jnp.where(K2 + zvec < 0.5, lo_f + 0.5 * w2, tau)

            def out_body(j, carry):
                v = row_v[pl.ds(j * L, L)]
                row_v[pl.ds(j * L, L)] = jnp.maximum(v - tau, 0.0)
                return carry
            lax.fori_loop(0, nvec, out_body, 0)

            pltpu.sync_copy(row_v, out_hbm.at[r])

        def row_loop(i, carry):
            process_row(wid * rows_per_w + i)
            return carry
        lax.fori_loop(0, rows_per_w, row_loop, 0)

    return sparsemax_kernel(x)


def kernel(x):
    return _sparsemax_rows(x)


# speedup vs baseline: 63.6289x; 10.0295x over previous
import functools

import jax
import jax.numpy as jnp
from jax import lax
from jax.experimental import pallas as pl
from jax.experimental.pallas import tpu as pltpu
from jax.experimental.pallas import tpu_sc as plsc

NC = 2
NS = 16
NW = NC * NS
L = 16
B = 1024


def _sparsemax_rows(x):
    R, N = x.shape
    nvec = N // L
    rows_per_w = R // NW
    nchunk = B // L

    mesh = plsc.VectorSubcoreMesh(core_axis_name="c", subcore_axis_name="s")

    @functools.partial(
        pl.kernel,
        out_type=jax.ShapeDtypeStruct((R, N), jnp.float32),
        mesh=mesh,
        scratch_types=[
            pltpu.VMEM((N,), jnp.float32),
            pltpu.VMEM((B,), jnp.float32),
            pltpu.VMEM((B,), jnp.float32),
        ],
        compiler_params=pltpu.CompilerParams(needs_layout_passes=False),
    )
    def sparsemax_kernel(x_hbm, out_hbm, row_v, cnt_v, sum_v):
        wid = lax.axis_index("s") * NC + lax.axis_index("c")
        zvec = jnp.zeros((L,), jnp.float32)
        ones = jnp.ones((L,), jnp.float32)
        iota_f = lax.iota(jnp.int32, L).astype(jnp.float32)
        rev_iota_f = jnp.float32(L - 1) - iota_f
        last_idx = jnp.full((L,), L - 1, jnp.int32)

        def zero_hist():
            @functools.partial(plsc.parallel_loop, 0, nchunk, unroll=4)
            def _(j):
                cnt_v[pl.ds(j * L, L)] = zvec
                sum_v[pl.ds(j * L, L)] = zvec

        def scan_round(K0, S0, lo, w):
            def body(t, carry):
                carryC, carryS, predsum, kaddv, saddv = carry
                cb = (nchunk - 1) - t
                c = cnt_v[pl.ds(cb * L, L)]
                s = sum_v[pl.ds(cb * L, L)]
                rc = lax.rev(c, (0,))
                rs = lax.rev(s, (0,))
                csum = plsc.cumsum(rc) + carryC
                ssum = plsc.cumsum(rs) + carryS
                base = (cb * L).astype(jnp.float32)
                edges = lo + w * (base + rev_iota_f)
                f = (S0 + ssum) - (K0 + csum) * edges
                pred = jnp.where(f >= 1.0, 1.0, 0.0)
                npred = 1.0 - pred
                return (csum.at[last_idx].get(mode="promise_in_bounds"),
                        ssum.at[last_idx].get(mode="promise_in_bounds"),
                        predsum + pred, kaddv + rc * npred,
                        saddv + rs * npred)

            init = (zvec, zvec, zvec, zvec, zvec)
            _, _, predsum, kaddv, saddv = lax.fori_loop(
                0, nchunk, body, init)
            jstar = jnp.maximum(jnp.sum(predsum) - 1.0, 0.0)
            return jstar, K0 + jnp.sum(kaddv), S0 + jnp.sum(saddv)

        def process_row(r):
            pltpu.sync_copy(x_hbm.at[r], row_v)

            def max_body(j, acc):
                return jnp.maximum(acc, row_v[pl.ds(j * L, L)])
            acc = plsc.parallel_loop(
                0, nvec, unroll=8,
                carry=jnp.full((L,), -jnp.inf, jnp.float32))(max_body)
            m = jnp.max(acc)

            lo1 = m - 1.0
            span1 = 1.0 + 1.0 / B
            inv_w1 = B / span1
            w1 = span1 / B
            zero_hist()

            @functools.partial(plsc.parallel_loop, 0, nvec, unroll=8)
            def _(j):
                v = row_v[pl.ds(j * L, L)]
                ji = ((v - lo1) * inv_w1).astype(jnp.int32)
                msk = v >= lo1
                plsc.addupdate_scatter(cnt_v, [ji], ones, mask=msk)
                plsc.addupdate_scatter(sum_v, [ji], v, mask=msk)

            jstar1, K1, S1 = scan_round(
                jnp.float32(0.0), jnp.float32(0.0), lo1, w1)

            lo2 = lo1 + jstar1 * w1
            hi2 = lo2 + w1
            inv_w2 = jnp.float32(B) / w1
            w2 = w1 / B
            zero_hist()

            @functools.partial(plsc.parallel_loop, 0, nvec, unroll=8)
            def _(j):
                v = row_v[pl.ds(j * L, L)]
                msk = (v >= lo2) & (v < hi2)
                j2 = jnp.clip(((v - lo2) * inv_w2).astype(jnp.int32),
                              0, B - 1)
                plsc.addupdate_scatter(cnt_v, [j2], ones, mask=msk)
                plsc.addupdate_scatter(sum_v, [j2], v, mask=msk)

            jstar2, K2, S2 = scan_round(K1, S1, lo2, w2)

            lo_f = lo2 + jstar2 * w2
            tau = jnp.clip((S2 + zvec - 1.0) / jnp.maximum(K2 + zvec, 1.0),
                           lo_f, lo_f + w2)
            tau = jnp.where(K2 + zvec < 0.5, lo_f + 0.5 * w2, tau)

            @functools.partial(plsc.parallel_loop, 0, nvec, unroll=8)
            def _(j):
                v = row_v[pl.ds(j * L, L)]
                row_v[pl.ds(j * L, L)] = jnp.maximum(v - tau, 0.0)

            pltpu.sync_copy(row_v, out_hbm.at[r])

        def row_loop(i, carry):
            process_row(wid * rows_per_w + i)
            return carry
        lax.fori_loop(0, rows_per_w, row_loop, 0)

    return sparsemax_kernel(x)


def kernel(x):
    return _sparsemax_rows(x)
